# baseline (speedup 1.0000x reference)
import jax
import jax.numpy as jnp
from jax.experimental import pallas as pl

S = 2048
D = 1024
H = 16
DH = 64
BS = 64
NB = 32
SCALE = 1.0 / (D ** 0.5)
NEG = -1e9


def _qkv_kernel(x_ref, w_ref, o_ref):
    o_ref[...] = jnp.dot(x_ref[...], w_ref[...].T,
                         preferred_element_type=jnp.float32)


def _mask_kernel(k_ref, w1_ref, b1_ref, w2_ref, b2_ref, o_ref):
    k = k_ref[...]
    kblk = k.reshape(NB, BS, D).mean(axis=1)
    h1 = jnp.maximum(
        jnp.dot(kblk, w1_ref[...].T, preferred_element_type=jnp.float32)
        + b1_ref[...], 0.0)
    w2b = jnp.broadcast_to(w2_ref[...], (BS, 128))
    s = jnp.dot(h1, w2b.T, preferred_element_type=jnp.float32) \
        + jnp.sum(b2_ref[...])
    keep = s > 0.0
    o_ref[...] = jnp.where(keep, 0.0, NEG).astype(jnp.float32)


def _attn_kernel(q_ref, k_ref, v_ref, cm_ref, o_ref):
    i = pl.program_id(1)
    q = q_ref[0] * SCALE
    keep_q = jnp.max(cm_ref[pl.ds(i, 1), :]) > -1.0
    row_ids = jax.lax.broadcasted_iota(jnp.int32, (BS, BS), 0)
    col_ids = jax.lax.broadcasted_iota(jnp.int32, (BS, BS), 1)

    def body(j, carry):
        acc, m, l = carry
        kb = k_ref[0, pl.ds(j * BS, BS), :]
        vb = v_ref[0, pl.ds(j * BS, BS), :]
        cmj = cm_ref[pl.ds(j, 1), :]
        s = jnp.dot(q, kb.T, preferred_element_type=jnp.float32)
        s = jnp.where(keep_q, s + cmj, NEG)
        s = jnp.where(jnp.logical_and(j == i, col_ids > row_ids),
                      -jnp.inf, s)
        m_new = jnp.maximum(m, jnp.max(s, axis=1))
        p = jnp.exp(s - m_new[:, None])
        alpha = jnp.exp(m - m_new)
        l_new = l * alpha + jnp.sum(p, axis=1)
        acc_new = acc * alpha[:, None] + jnp.dot(
            p, vb, preferred_element_type=jnp.float32)
        return acc_new, m_new, l_new

    def step(j, carry):
        blk_dropped = jnp.max(cm_ref[pl.ds(j, 1), :]) < -1.0
        skip = jnp.logical_and(keep_q, blk_dropped)
        return jax.lax.cond(skip, lambda c: c, lambda c: body(j, c), carry)

    init = (jnp.zeros((BS, DH), jnp.float32),
            jnp.full((BS,), -1e30, jnp.float32),
            jnp.zeros((BS,), jnp.float32))
    acc, m, l = jax.lax.fori_loop(0, i + 1, step, init)
    o_ref[0] = acc / l[:, None]


def _proj_kernel(a_ref, w_ref, b_ref, o_ref):
    o_ref[...] = jnp.dot(a_ref[...], w_ref[...].T,
                         preferred_element_type=jnp.float32) + b_ref[...]


def kernel(x, Wqkv, rW1, rb1, rW2, rb2, Wo, bo):
    x2 = x[0]
    qkv = pl.pallas_call(
        _qkv_kernel,
        grid=(6,),
        in_specs=[pl.BlockSpec((S, D), lambda j: (0, 0)),
                  pl.BlockSpec((512, D), lambda j: (j, 0))],
        out_specs=pl.BlockSpec((S, 512), lambda j: (0, j)),
        out_shape=jax.ShapeDtypeStruct((S, 3 * D), jnp.float32),
    )(x2, Wqkv)
    q, k, v = qkv[:, :D], qkv[:, D:2 * D], qkv[:, 2 * D:]
    cm = pl.pallas_call(
        _mask_kernel,
        out_shape=jax.ShapeDtypeStruct((NB, BS), jnp.float32),
    )(k, rW1, rb1.reshape(1, 128), rW2, rb2.reshape(1, 1))
    qh = q.reshape(S, H, DH).transpose(1, 0, 2)
    kh = k.reshape(S, H, DH).transpose(1, 0, 2)
    vh = v.reshape(S, H, DH).transpose(1, 0, 2)
    ao = pl.pallas_call(
        _attn_kernel,
        grid=(H, NB),
        in_specs=[pl.BlockSpec((1, BS, DH), lambda h, i: (h, i, 0)),
                  pl.BlockSpec((1, S, DH), lambda h, i: (h, 0, 0)),
                  pl.BlockSpec((1, S, DH), lambda h, i: (h, 0, 0)),
                  pl.BlockSpec((NB, BS), lambda h, i: (0, 0))],
        out_specs=pl.BlockSpec((1, BS, DH), lambda h, i: (h, i, 0)),
        out_shape=jax.ShapeDtypeStruct((H, S, DH), jnp.float32),
    )(qh, kh, vh, cm)
    a = ao.transpose(1, 0, 2).reshape(S, D)
    out = pl.pallas_call(
        _proj_kernel,
        grid=(4,),
        in_specs=[pl.BlockSpec((S, D), lambda j: (0, 0)),
                  pl.BlockSpec((256, D), lambda j: (j, 0)),
                  pl.BlockSpec((1, 256), lambda j: (0, j))],
        out_specs=pl.BlockSpec((S, 256), lambda j: (0, j)),
        out_shape=jax.ShapeDtypeStruct((S, D), jnp.float32),
    )(a, Wo, bo.reshape(1, D))
    return out[None], kh[None], vh[None]


# speedup vs baseline: 10.4792x; 10.4792x over previous
import jax
import jax.numpy as jnp
from jax.experimental import pallas as pl
from jax.experimental.pallas import tpu as pltpu

S = 2048
D = 1024
H = 16
DH = 64
BS = 64
NB = 32
BQ = 256
BK = 512
NQ = S // BQ
NCH = S // BK
SCALE = 1.0 / (D ** 0.5)
NEG = -1e9


def _qkv_kernel(x_ref, w_ref, o_ref):
    r = jnp.dot(x_ref[...], w_ref[...].T, preferred_element_type=jnp.float32)
    for t in range(8):
        o_ref[t] = r[:, t * DH:(t + 1) * DH]


def _mask_kernel(kh_ref, w1_ref, b1_ref, w2_ref, b2_ref, o_ref):
    h1 = b1_ref[...]
    for h in range(H):
        kblk = kh_ref[h].reshape(NB, BS, DH).mean(axis=1)
        w1h = w1_ref[:, h * DH:(h + 1) * DH]
        h1 = h1 + jnp.dot(kblk, w1h.T, preferred_element_type=jnp.float32)
    h1 = jnp.maximum(h1, 0.0)
    w2b = jnp.broadcast_to(w2_ref[...], (BS, 128))
    s = jnp.dot(h1, w2b.T, preferred_element_type=jnp.float32) \
        + jnp.sum(b2_ref[...])
    keep = s > 0.0
    o_ref[...] = jnp.where(keep, 0.0, NEG).astype(jnp.float32)


def _attn_kernel(mask_ref, q_ref, k_ref, v_ref, cm_ref, o_ref):
    qi = pl.program_id(1)
    q = q_ref[0] * SCALE

    ri = jax.lax.broadcasted_iota(jnp.int32, (BQ, BK), 0) // BS
    rk = jnp.zeros((BQ, BK), jnp.float32)
    for t in range(BQ // BS):
        rk = rk + jnp.where(ri == t,
                            mask_ref[4 * qi + t].astype(jnp.float32), 0.0)

    rows = qi * BQ + jax.lax.broadcasted_iota(jnp.int32, (BQ, BK), 0)
    cols_local = jax.lax.broadcasted_iota(jnp.int32, (BQ, BK), 1)

    def body(j, carry):
        acc, m, l = carry
        kb = k_ref[0, pl.ds(j * BK, BK), :]
        vb = v_ref[0, pl.ds(j * BK, BK), :]
        cmj = cm_ref[pl.ds(j, 1), :]
        s = jnp.dot(q, kb.T, preferred_element_type=jnp.float32)
        s = rk * (s + cmj)
        s = jnp.where(j * BK + cols_local > rows, -jnp.inf, s)
        m_new = jnp.maximum(m, jnp.max(s, axis=1, keepdims=True))
        p = jnp.exp(s - m_new)
        alpha = jnp.exp(m - m_new)
        l_new = l * alpha + jnp.sum(p, axis=1, keepdims=True)
        acc_new = acc * alpha + jnp.dot(p, vb,
                                        preferred_element_type=jnp.float32)
        return acc_new, m_new, l_new

    init = (jnp.zeros((BQ, DH), jnp.float32),
            jnp.full((BQ, 1), -1e30, jnp.float32),
            jnp.zeros((BQ, 1), jnp.float32))
    nch = qi * BQ // BK + 1
    acc, m, l = jax.lax.fori_loop(0, nch, body, init)
    o_ref[0] = acc / l


def _proj_kernel(a_ref, w_ref, b_ref, o_ref):
    acc = jnp.broadcast_to(b_ref[...], (BQ, D))
    for h in range(H):
        wh = w_ref[:, h * DH:(h + 1) * DH]
        acc = acc + jnp.dot(a_ref[h], wh.T, preferred_element_type=jnp.float32)
    o_ref[...] = acc


def kernel(x, Wqkv, rW1, rb1, rW2, rb2, Wo, bo):
    x2 = x[0]
    qkvh = pl.pallas_call(
        _qkv_kernel,
        grid=(6,),
        in_specs=[pl.BlockSpec((S, D), lambda j: (0, 0)),
                  pl.BlockSpec((512, D), lambda j: (j, 0))],
        out_specs=pl.BlockSpec((8, S, DH), lambda j: (j, 0, 0)),
        out_shape=jax.ShapeDtypeStruct((3 * H, S, DH), jnp.float32),
    )(x2, Wqkv)
    qh, kh, vh = qkvh[:H], qkvh[H:2 * H], qkvh[2 * H:]
    cm = pl.pallas_call(
        _mask_kernel,
        grid=(1,),
        in_specs=[pl.BlockSpec((H, S, DH), lambda i: (1, 0, 0)),
                  pl.BlockSpec((128, D), lambda i: (0, 0)),
                  pl.BlockSpec((1, 128), lambda i: (0, 0)),
                  pl.BlockSpec((1, 128), lambda i: (0, 0)),
                  pl.BlockSpec((1, 1), lambda i: (0, 0))],
        out_specs=pl.BlockSpec((NB, BS), lambda i: (0, 0)),
        out_shape=jax.ShapeDtypeStruct((NB, BS), jnp.float32),
    )(qkvh, rW1, rb1.reshape(1, 128), rW2, rb2.reshape(1, 1))
    mask_i32 = (cm[:, 0] > -1.0).astype(jnp.int32)
    cmk = cm.reshape(NCH, BK)
    ao = pl.pallas_call(
        _attn_kernel,
        grid=(H, NQ),
        in_specs=[pl.BlockSpec(memory_space=pltpu.SMEM),
                  pl.BlockSpec((1, BQ, DH), lambda h, i: (h, i, 0)),
                  pl.BlockSpec((1, S, DH), lambda h, i: (h, 0, 0)),
                  pl.BlockSpec((1, S, DH), lambda h, i: (h, 0, 0)),
                  pl.BlockSpec((NCH, BK), lambda h, i: (0, 0))],
        out_specs=pl.BlockSpec((1, BQ, DH), lambda h, i: (h, i, 0)),
        out_shape=jax.ShapeDtypeStruct((H, S, DH), jnp.float32),
    )(mask_i32, qh, kh, vh, cmk)
    out = pl.pallas_call(
        _proj_kernel,
        grid=(NQ,),
        in_specs=[pl.BlockSpec((H, BQ, DH), lambda i: (0, i, 0)),
                  pl.BlockSpec((D, D), lambda i: (0, 0)),
                  pl.BlockSpec((1, D), lambda i: (0, 0))],
        out_specs=pl.BlockSpec((BQ, D), lambda i: (i, 0)),
        out_shape=jax.ShapeDtypeStruct((S, D), jnp.float32),
    )(ao, Wo, bo.reshape(1, D))
    return out[None], kh[None], vh[None]
